# Initial kernel scaffold; baseline (speedup 1.0000x reference)
import jax
import jax.numpy as jnp
from jax.experimental import pallas as pl


def kernel(idx, nodeData, node_inverse, node_dst_inverse, efeat_unique, efeat_inverse, time_unique, time_inverse, time_dst_unique, time_dst_inverse, W_q_node, b_q_node, W_q_time, b_q_time, W_kv_node, b_kv_node, W_kv_edge, b_kv_edge, W_kv_time, b_kv_time):
    raise NotImplementedError("write your pallas kernel here")


# speedup vs baseline: 2.5112x; 2.5112x over previous
import dataclasses
import functools

import jax
import jax.numpy as jnp
from jax.experimental import pallas as pl
from jax.experimental.pallas import tpu as pltpu
from jax.experimental.pallas import tpu_sc as plsc

N = 10000
E = 320000
D_OUT = 128
H = 8

_NP = 10240
_CQ = 128
_CE = 128
_CH = 64

_vector_mesh = plsc.VectorSubcoreMesh(
    core_axis_name="core", subcore_axis_name="subcore")

_sc_params = pltpu.CompilerParams()
if "needs_layout_passes" in pltpu.CompilerParams.__dataclass_fields__:
  _sc_params = dataclasses.replace(_sc_params, needs_layout_passes=False)


def _linear(x, w, b, block_rows=512):
  R, K = x.shape
  Dout = w.shape[1]

  def body(x_ref, w_ref, b_ref, o_ref):
    o_ref[...] = jnp.dot(x_ref[...], w_ref[...],
                         preferred_element_type=jnp.float32) + b_ref[...]

  br = min(block_rows, R)
  return pl.pallas_call(
      body,
      grid=(pl.cdiv(R, br),),
      in_specs=[
          pl.BlockSpec((br, K), lambda i: (i, 0)),
          pl.BlockSpec((K, Dout), lambda i: (0, 0)),
          pl.BlockSpec((1, Dout), lambda i: (0, 0)),
      ],
      out_specs=pl.BlockSpec((br, Dout), lambda i: (i, 0)),
      out_shape=jax.ShapeDtypeStruct((R, Dout), jnp.float32),
  )(x, w, b.reshape(1, Dout))


def _qour_sc(ndi, tdi, qn, qt):
  @functools.partial(
      pl.kernel,
      out_type=jax.ShapeDtypeStruct((_NP, D_OUT), jnp.float32),
      mesh=_vector_mesh,
      scratch_types=[pltpu.VMEM((_CQ, D_OUT), jnp.float32)],
      compiler_params=_sc_params,
  )
  def k(ndi_hbm, tdi_hbm, qn_hbm, qt_hbm, out_hbm, qt_s):
    def body(ndi_v, tdi_v, out_v):
      pltpu.sync_copy(qn_hbm.at[ndi_v.at[0]], out_v)
      pltpu.sync_copy(qt_hbm.at[tdi_v.at[0]], qt_s)

      @pl.loop(0, _CQ)
      def _(r):
        for j in range(D_OUT // 16):
          sl = pl.ds(j * 16, 16)
          out_v[r, sl] += qt_s[r, sl]

    pltpu.emit_pipeline(
        body,
        grid=(_NP // _CQ,),
        in_specs=[
            pl.BlockSpec((1, _CQ), lambda i: (0, i)),
            pl.BlockSpec((1, _CQ), lambda i: (0, i)),
        ],
        out_specs=[pl.BlockSpec((_CQ, D_OUT), lambda i: (i, 0))],
        core_axis_name=("core", "subcore"),
        dimension_semantics=(pltpu.PARALLEL,),
    )(ndi_hbm, tdi_hbm, out_hbm)

  return k(ndi, tdi, qn, qt)


def _edge_sc(idx2, ni2, ei2, ti2, qour, zn, ze, zt):
  @functools.partial(
      pl.kernel,
      out_type=(jax.ShapeDtypeStruct((E // 2, 2 * H), jnp.float32),
                jax.ShapeDtypeStruct((E, D_OUT), jnp.float32)),
      mesh=_vector_mesh,
      scratch_types=[
          pltpu.VMEM((_CH, D_OUT), jnp.float32),
          pltpu.VMEM((_CH, 2 * D_OUT), jnp.float32),
          pltpu.VMEM((_CH, 2 * D_OUT), jnp.float32),
          pltpu.VMEM((_CH, 2 * D_OUT), jnp.float32),
      ],
      compiler_params=_sc_params,
  )
  def k(idx_hbm, ni_hbm, ei_hbm, ti_hbm, qour_hbm, zn_hbm, ze_hbm, zt_hbm,
        attn_hbm, v_hbm, q_s, zn_s, ze_s, zt_s):
    def body(idx_v, ni_v, ei_v, ti_v, attn_v, v_v):
      lane = jax.lax.iota(jnp.int32, 16)
      for half in range(_CE // _CH):
        isl = pl.ds(half * _CH, _CH)
        pltpu.sync_copy(qour_hbm.at[idx_v.at[0, isl]], q_s)
        pltpu.sync_copy(zn_hbm.at[ni_v.at[0, isl]], zn_s)
        pltpu.sync_copy(ze_hbm.at[ei_v.at[0, isl]], ze_s)
        pltpu.sync_copy(zt_hbm.at[ti_v.at[0, isl]], zt_s)

        @pl.loop(0, _CH // 2)
        def _(i, half=half):
          e0 = 2 * i
          e1 = 2 * i + 1
          acc = jnp.zeros((16,), jnp.float32)
          for h in range(H):
            sk = pl.ds(h * 16, 16)
            sv = pl.ds(D_OUT + h * 16, 16)
            k0 = zn_s[e0, sk] + ze_s[e0, sk] + zt_s[e0, sk]
            s0 = jnp.sum(q_s[e0, sk] * k0)
            k1 = zn_s[e1, sk] + ze_s[e1, sk] + zt_s[e1, sk]
            s1 = jnp.sum(q_s[e1, sk] * k1)
            acc = jnp.where(lane == h, s0, acc)
            acc = jnp.where(lane == H + h, s1, acc)
            v_v[half * _CH + e0, sk] = zn_s[e0, sv] + ze_s[e0, sv] + zt_s[e0, sv]
            v_v[half * _CH + e1, sk] = zn_s[e1, sv] + ze_s[e1, sv] + zt_s[e1, sv]
          attn_v[half * (_CH // 2) + i, :] = jnp.where(acc >= 0., acc, 0.2 * acc)

    pltpu.emit_pipeline(
        body,
        grid=(E // _CE,),
        in_specs=[pl.BlockSpec((1, _CE), lambda i: (0, i))] * 4,
        out_specs=[
            pl.BlockSpec((_CE // 2, 2 * H), lambda i: (i, 0)),
            pl.BlockSpec((_CE, D_OUT), lambda i: (i, 0)),
        ],
        core_axis_name=("core", "subcore"),
        dimension_semantics=(pltpu.PARALLEL,),
    )(idx_hbm, ni_hbm, ei_hbm, ti_hbm, attn_hbm, v_hbm)

  return k(idx2, ni2, ei2, ti2, qour, zn, ze, zt)


def kernel(idx, nodeData, node_inverse, node_dst_inverse, efeat_unique,
           efeat_inverse, time_unique, time_inverse, time_dst_unique,
           time_dst_inverse, W_q_node, b_q_node, W_q_time, b_q_time,
           W_kv_node, b_kv_node, W_kv_edge, b_kv_edge, W_kv_time, b_kv_time):
  qn = _linear(nodeData, W_q_node, b_q_node)
  qt = _linear(time_dst_unique, W_q_time, b_q_time)
  znode = _linear(nodeData, W_kv_node, b_kv_node)
  zedge = _linear(efeat_unique, W_kv_edge, b_kv_edge)
  ztime = _linear(time_unique, W_kv_time, b_kv_time)

  pad = _NP - N
  ndi = jnp.pad(node_dst_inverse, (0, pad)).reshape(1, _NP)
  tdi = jnp.pad(time_dst_inverse, (0, pad)).reshape(1, _NP)
  qour = _qour_sc(ndi, tdi, qn, qt)

  attn2, v = _edge_sc(
      idx.reshape(1, E), node_inverse.reshape(1, E),
      efeat_inverse.reshape(1, E), time_inverse.reshape(1, E),
      qour, znode, zedge, ztime)
  return (attn2.reshape(E, H), v.reshape(E, H, D_OUT // H))


# speedup vs baseline: 4.7538x; 1.8930x over previous
import dataclasses
import functools

import jax
import jax.numpy as jnp
from jax.experimental import pallas as pl
from jax.experimental.pallas import tpu as pltpu
from jax.experimental.pallas import tpu_sc as plsc

N = 10000
E = 320000
D_OUT = 128
H = 8

_NP = 10240
_CQ = 128
_CE = 128
_CH = 64

_vector_mesh = plsc.VectorSubcoreMesh(
    core_axis_name="core", subcore_axis_name="subcore")

_sc_params = pltpu.CompilerParams()
if "needs_layout_passes" in pltpu.CompilerParams.__dataclass_fields__:
  _sc_params = dataclasses.replace(_sc_params, needs_layout_passes=False)


def _linear(x, w, b, block_rows=512):
  R, K = x.shape
  Dout = w.shape[1]

  def body(x_ref, w_ref, b_ref, o_ref):
    o_ref[...] = jnp.dot(x_ref[...], w_ref[...],
                         preferred_element_type=jnp.float32) + b_ref[...]

  br = min(block_rows, R)
  return pl.pallas_call(
      body,
      grid=(pl.cdiv(R, br),),
      in_specs=[
          pl.BlockSpec((br, K), lambda i: (i, 0)),
          pl.BlockSpec((K, Dout), lambda i: (0, 0)),
          pl.BlockSpec((1, Dout), lambda i: (0, 0)),
      ],
      out_specs=pl.BlockSpec((br, Dout), lambda i: (i, 0)),
      out_shape=jax.ShapeDtypeStruct((R, Dout), jnp.float32),
  )(x, w, b.reshape(1, Dout))


def _qour_sc(ndi, tdi, qn, qt):
  @functools.partial(
      pl.kernel,
      out_type=jax.ShapeDtypeStruct((_NP, D_OUT), jnp.float32),
      mesh=_vector_mesh,
      scratch_types=[pltpu.VMEM((_CQ, D_OUT), jnp.float32)],
      compiler_params=_sc_params,
  )
  def k(ndi_hbm, tdi_hbm, qn_hbm, qt_hbm, out_hbm, qt_s):
    def body(ndi_v, tdi_v, out_v):
      pltpu.sync_copy(qn_hbm.at[ndi_v.at[0]], out_v)
      pltpu.sync_copy(qt_hbm.at[tdi_v.at[0]], qt_s)

      @pl.loop(0, _CQ)
      def _(r):
        for j in range(D_OUT // 16):
          sl = pl.ds(j * 16, 16)
          out_v[r, sl] += qt_s[r, sl]

    pltpu.emit_pipeline(
        body,
        grid=(_NP // _CQ,),
        in_specs=[
            pl.BlockSpec((1, _CQ), lambda i: (0, i)),
            pl.BlockSpec((1, _CQ), lambda i: (0, i)),
        ],
        out_specs=[pl.BlockSpec((_CQ, D_OUT), lambda i: (i, 0))],
        core_axis_name=("core", "subcore"),
        dimension_semantics=(pltpu.PARALLEL,),
    )(ndi_hbm, tdi_hbm, out_hbm)

  return k(ndi, tdi, qn, qt)


_NW = 32
_PW = E // _NW
_C = 40
_SC = 2000
_NS = _PW // _SC
_JC = _SC // _C


def _edge_sc(idx1, ni1, ei1, ti1, qour, zn, ze, zt):
  @functools.partial(
      pl.kernel,
      out_type=(jax.ShapeDtypeStruct((E, H), jnp.float32),
                jax.ShapeDtypeStruct((E, D_OUT), jnp.float32)),
      mesh=_vector_mesh,
      scratch_types=[
          [pltpu.VMEM((_SC,), jnp.int32) for _ in range(4)],
          [pltpu.VMEM((_C, D_OUT), jnp.float32) for _ in range(2)],
          [pltpu.VMEM((_C, 2 * D_OUT), jnp.float32) for _ in range(2)],
          [pltpu.VMEM((_C, 2 * D_OUT), jnp.float32) for _ in range(2)],
          [pltpu.VMEM((_C, 2 * D_OUT), jnp.float32) for _ in range(2)],
          [pltpu.VMEM((_C, D_OUT), jnp.float32) for _ in range(2)],
          [pltpu.VMEM((_C, H), jnp.float32) for _ in range(2)],
          [pltpu.SemaphoreType.DMA for _ in range(2)],
          [pltpu.SemaphoreType.DMA for _ in range(2)],
          pltpu.SemaphoreType.DMA,
      ],
      compiler_params=_sc_params,
  )
  def k(idx_hbm, ni_hbm, ei_hbm, ti_hbm, qour_hbm, zn_hbm, ze_hbm, zt_hbm,
        attn_hbm, v_hbm, ib, qb, znb, zeb, ztb, vb, ab, gsem, osem, isem):
    w = jax.lax.axis_index("subcore") * 2 + jax.lax.axis_index("core")
    base = w * _PW

    def gathers(j, b):
      sl = pl.ds(j * _C, _C)
      return [
          pltpu.make_async_copy(qour_hbm.at[ib[0].at[sl]], qb[b], gsem[b]),
          pltpu.make_async_copy(zn_hbm.at[ib[1].at[sl]], znb[b], gsem[b]),
          pltpu.make_async_copy(ze_hbm.at[ib[2].at[sl]], zeb[b], gsem[b]),
          pltpu.make_async_copy(zt_hbm.at[ib[3].at[sl]], ztb[b], gsem[b]),
      ]

    def outs(c, b):
      sl = pl.ds(base + c * _C, _C)
      return [
          pltpu.make_async_copy(vb[b], v_hbm.at[sl], osem[b]),
          pltpu.make_async_copy(ab[b], attn_hbm.at[sl], osem[b]),
      ]

    lane = jax.lax.iota(jnp.int32, 16)
    prow = lane >> 3
    pcol = lane & 7

    @pl.loop(0, _NS)
    def _(s):
      for src, dst in zip((idx_hbm, ni_hbm, ei_hbm, ti_hbm), ib):
        pltpu.async_copy(src.at[pl.ds(base + s * _SC, _SC)], dst, isem)
      for src, dst in zip((idx_hbm, ni_hbm, ei_hbm, ti_hbm), ib):
        pltpu.make_async_copy(src.at[pl.ds(base + s * _SC, _SC)], dst, isem).wait()

      for d in gathers(0, 0):
        d.start()

      @pl.loop(0, _JC, step=2)
      def _(j0, s=s):
        for b in range(2):
          j = j0 + b
          c = s * _JC + j

          @pl.when(j + 1 < _JC)
          def _(j=j, b=b):
            for d in gathers(j + 1, 1 - b):
              d.start()

          for d in gathers(j, b):
            d.wait()

          @pl.when(c >= 2)
          def _(c=c, b=b):
            for d in outs(c - 2, b):
              d.wait()

          q_s, zn_s, ze_s, zt_s, v_s, a_s = qb[b], znb[b], zeb[b], ztb[b], vb[b], ab[b]

          @pl.loop(0, _C // 2)
          def _(i, q_s=q_s, zn_s=zn_s, ze_s=ze_s, zt_s=zt_s, v_s=v_s, a_s=a_s):
            e0 = 2 * i
            e1 = 2 * i + 1
            acc = jnp.zeros((16,), jnp.float32)
            for h in range(H):
              sk = pl.ds(h * 16, 16)
              sv = pl.ds(D_OUT + h * 16, 16)
              k0 = zn_s[e0, sk] + ze_s[e0, sk] + zt_s[e0, sk]
              s0 = jnp.sum(q_s[e0, sk] * k0)
              k1 = zn_s[e1, sk] + ze_s[e1, sk] + zt_s[e1, sk]
              s1 = jnp.sum(q_s[e1, sk] * k1)
              acc = jnp.where(lane == h, s0, acc)
              acc = jnp.where(lane == H + h, s1, acc)
              v_s[e0, sk] = zn_s[e0, sv] + ze_s[e0, sv] + zt_s[e0, sv]
              v_s[e1, sk] = zn_s[e1, sv] + ze_s[e1, sv] + zt_s[e1, sv]
            acc = jnp.where(acc >= 0., acc, 0.2 * acc)
            plsc.store_scatter(a_s, [2 * i + prow, pcol], acc)

          for d in outs(c, b):
            d.start()

    for b in range(2):
      for d in outs(_NS * _JC - 2 + b, b):
        d.wait()

  return k(idx1, ni1, ei1, ti1, qour, zn, ze, zt)


def kernel(idx, nodeData, node_inverse, node_dst_inverse, efeat_unique,
           efeat_inverse, time_unique, time_inverse, time_dst_unique,
           time_dst_inverse, W_q_node, b_q_node, W_q_time, b_q_time,
           W_kv_node, b_kv_node, W_kv_edge, b_kv_edge, W_kv_time, b_kv_time):
  qn = _linear(nodeData, W_q_node, b_q_node)
  qt = _linear(time_dst_unique, W_q_time, b_q_time)
  znode = _linear(nodeData, W_kv_node, b_kv_node)
  zedge = _linear(efeat_unique, W_kv_edge, b_kv_edge)
  ztime = _linear(time_unique, W_kv_time, b_kv_time)

  pad = _NP - N
  ndi = jnp.pad(node_dst_inverse, (0, pad)).reshape(1, _NP)
  tdi = jnp.pad(time_dst_inverse, (0, pad)).reshape(1, _NP)
  qour = _qour_sc(ndi, tdi, qn, qt)

  attn, v = _edge_sc(
      idx, node_inverse, efeat_inverse, time_inverse,
      qour, znode, zedge, ztime)
  return (attn, v.reshape(E, H, D_OUT // H))


# speedup vs baseline: 4.8910x; 1.0289x over previous
import dataclasses
import functools

import jax
import jax.numpy as jnp
import numpy as np
from jax.experimental import pallas as pl
from jax.experimental.pallas import tpu as pltpu
from jax.experimental.pallas import tpu_sc as plsc

N = 10000
E = 320000
D_OUT = 128
H = 8

_NP = 10240
_CQ = 128
_CE = 128
_CH = 64

_vector_mesh = plsc.VectorSubcoreMesh(
    core_axis_name="core", subcore_axis_name="subcore")

_sc_params = pltpu.CompilerParams()
if "needs_layout_passes" in pltpu.CompilerParams.__dataclass_fields__:
  _sc_params = dataclasses.replace(_sc_params, needs_layout_passes=False)


def _interleave_perm(width):
  perm = np.empty((width,), np.int32)
  for j in range(width // 32):
    for t in range(16):
      perm[32 * j + 2 * t] = 32 * j + t
      perm[32 * j + 2 * t + 1] = 32 * j + 16 + t
  return perm


_PERM128 = _interleave_perm(D_OUT)
_PERM256 = _interleave_perm(2 * D_OUT)


def _toi32(x):
  r, d = x.shape
  return jax.lax.bitcast_convert_type(x.reshape(r, d // 2, 2), jnp.int32)


def _linear(x, w, b, out_dtype=jnp.float32, block_rows=512):
  R, K = x.shape
  Dout = w.shape[1]

  def body(x_ref, w_ref, b_ref, o_ref):
    o_ref[...] = (jnp.dot(x_ref[...], w_ref[...],
                          preferred_element_type=jnp.float32)
                  + b_ref[...]).astype(out_dtype)

  br = min(block_rows, R)
  return pl.pallas_call(
      body,
      grid=(pl.cdiv(R, br),),
      in_specs=[
          pl.BlockSpec((br, K), lambda i: (i, 0)),
          pl.BlockSpec((K, Dout), lambda i: (0, 0)),
          pl.BlockSpec((1, Dout), lambda i: (0, 0)),
      ],
      out_specs=pl.BlockSpec((br, Dout), lambda i: (i, 0)),
      out_shape=jax.ShapeDtypeStruct((R, Dout), out_dtype),
  )(x, w, b.reshape(1, Dout))


def _qour_sc(ndi, tdi, qn, qt):
  @functools.partial(
      pl.kernel,
      out_type=jax.ShapeDtypeStruct((_NP, D_OUT), jnp.float32),
      mesh=_vector_mesh,
      scratch_types=[pltpu.VMEM((_CQ, D_OUT), jnp.float32)],
      compiler_params=_sc_params,
  )
  def k(ndi_hbm, tdi_hbm, qn_hbm, qt_hbm, out_hbm, qt_s):
    def body(ndi_v, tdi_v, out_v):
      pltpu.sync_copy(qn_hbm.at[ndi_v.at[0]], out_v)
      pltpu.sync_copy(qt_hbm.at[tdi_v.at[0]], qt_s)

      @pl.loop(0, _CQ)
      def _(r):
        for j in range(D_OUT // 16):
          sl = pl.ds(j * 16, 16)
          out_v[r, sl] += qt_s[r, sl]

    pltpu.emit_pipeline(
        body,
        grid=(_NP // _CQ,),
        in_specs=[
            pl.BlockSpec((1, _CQ), lambda i: (0, i)),
            pl.BlockSpec((1, _CQ), lambda i: (0, i)),
        ],
        out_specs=[pl.BlockSpec((_CQ, D_OUT), lambda i: (i, 0))],
        core_axis_name=("core", "subcore"),
        dimension_semantics=(pltpu.PARALLEL,),
    )(ndi_hbm, tdi_hbm, out_hbm)

  return k(ndi, tdi, qn, qt)


_NW = 32
_PW = E // _NW
_C = 40
_SC = 2000
_NS = _PW // _SC
_JC = _SC // _C


def _edge_sc(idx1, ni1, ei1, ti1, qour, zn, ze, zt):
  @functools.partial(
      pl.kernel,
      out_type=(jax.ShapeDtypeStruct((E, H), jnp.float32),
                jax.ShapeDtypeStruct((E, D_OUT), jnp.float32)),
      mesh=_vector_mesh,
      scratch_types=[
          [pltpu.VMEM((_SC,), jnp.int32) for _ in range(4)],
          [pltpu.VMEM((_C, D_OUT), jnp.float32) for _ in range(2)],
          [pltpu.VMEM((_C, D_OUT), jnp.int32) for _ in range(2)],
          [pltpu.VMEM((_C, D_OUT), jnp.int32) for _ in range(2)],
          [pltpu.VMEM((_C, D_OUT), jnp.int32) for _ in range(2)],
          [pltpu.VMEM((_C, D_OUT), jnp.float32) for _ in range(2)],
          [pltpu.VMEM((_C, H), jnp.float32) for _ in range(2)],
          [pltpu.SemaphoreType.DMA for _ in range(2)],
          [pltpu.SemaphoreType.DMA for _ in range(2)],
          pltpu.SemaphoreType.DMA,
      ],
      compiler_params=_sc_params,
  )
  def k(idx_hbm, ni_hbm, ei_hbm, ti_hbm, qour_hbm, zn_hbm, ze_hbm, zt_hbm,
        attn_hbm, v_hbm, ib, qb, znb, zeb, ztb, vb, ab, gsem, osem, isem):
    w = jax.lax.axis_index("subcore") * 2 + jax.lax.axis_index("core")
    base = w * _PW

    def gathers(j, b):
      sl = pl.ds(j * _C, _C)
      return [
          pltpu.make_async_copy(qour_hbm.at[ib[0].at[sl]], qb[b], gsem[b]),
          pltpu.make_async_copy(zn_hbm.at[ib[1].at[sl]], znb[b], gsem[b]),
          pltpu.make_async_copy(ze_hbm.at[ib[2].at[sl]], zeb[b], gsem[b]),
          pltpu.make_async_copy(zt_hbm.at[ib[3].at[sl]], ztb[b], gsem[b]),
      ]

    def outs(c, b):
      sl = pl.ds(base + c * _C, _C)
      return [
          pltpu.make_async_copy(vb[b], v_hbm.at[sl], osem[b]),
          pltpu.make_async_copy(ab[b], attn_hbm.at[sl], osem[b]),
      ]

    lane = jax.lax.iota(jnp.int32, 16)
    prow = lane >> 3
    pcol = lane & 7

    @pl.loop(0, _NS)
    def _(s):
      for src, dst in zip((idx_hbm, ni_hbm, ei_hbm, ti_hbm), ib):
        pltpu.async_copy(src.at[pl.ds(base + s * _SC, _SC)], dst, isem)
      for src, dst in zip((idx_hbm, ni_hbm, ei_hbm, ti_hbm), ib):
        pltpu.make_async_copy(src.at[pl.ds(base + s * _SC, _SC)], dst, isem).wait()

      for d in gathers(0, 0):
        d.start()

      @pl.loop(0, _JC, step=2)
      def _(j0, s=s):
        for b in range(2):
          j = j0 + b
          c = s * _JC + j

          @pl.when(j + 1 < _JC)
          def _(j=j, b=b):
            for d in gathers(j + 1, 1 - b):
              d.start()

          for d in gathers(j, b):
            d.wait()

          @pl.when(c >= 2)
          def _(c=c, b=b):
            for d in outs(c - 2, b):
              d.wait()

          q_s, zn_s, ze_s, zt_s, v_s, a_s = qb[b], znb[b], zeb[b], ztb[b], vb[b], ab[b]

          def unp(x):
            return plsc.unpack(plsc.bitcast(x, jnp.bfloat16),
                               format=plsc.PackFormat.INTERLEAVED)

          @pl.loop(0, _C // 2)
          def _(i, q_s=q_s, zn_s=zn_s, ze_s=ze_s, zt_s=zt_s, v_s=v_s, a_s=a_s):
            acc = jnp.zeros((16,), jnp.float32)
            for e, hbase in ((2 * i, 0), (2 * i + 1, H)):
              for j in range(D_OUT // 32):
                sk = pl.ds(16 * j, 16)
                qa = q_s[e, pl.ds(32 * j, 16)]
                qc = q_s[e, pl.ds(32 * j + 16, 16)]
                na, nc = unp(zn_s[e, sk])
                ea, ec = unp(ze_s[e, sk])
                ta, tc = unp(zt_s[e, sk])
                s0 = jnp.sum(qa * (na + ea + ta))
                s1 = jnp.sum(qc * (nc + ec + tc))
                acc = jnp.where(lane == hbase + 2 * j, s0, acc)
                acc = jnp.where(lane == hbase + 2 * j + 1, s1, acc)
              for j in range(D_OUT // 32):
                sv = pl.ds(D_OUT // 2 + 16 * j, 16)
                na, nc = unp(zn_s[e, sv])
                ea, ec = unp(ze_s[e, sv])
                ta, tc = unp(zt_s[e, sv])
                v_s[e, pl.ds(32 * j, 16)] = na + ea + ta
                v_s[e, pl.ds(32 * j + 16, 16)] = nc + ec + tc
            acc = jnp.where(acc >= 0., acc, 0.2 * acc)
            plsc.store_scatter(a_s, [2 * i + prow, pcol], acc)

          for d in outs(c, b):
            d.start()

    for b in range(2):
      for d in outs(_NS * _JC - 2 + b, b):
        d.wait()

  return k(idx1, ni1, ei1, ti1, qour, zn, ze, zt)


def kernel(idx, nodeData, node_inverse, node_dst_inverse, efeat_unique,
           efeat_inverse, time_unique, time_inverse, time_dst_unique,
           time_dst_inverse, W_q_node, b_q_node, W_q_time, b_q_time,
           W_kv_node, b_kv_node, W_kv_edge, b_kv_edge, W_kv_time, b_kv_time):
  qn = _linear(nodeData, W_q_node, b_q_node)
  qt = _linear(time_dst_unique, W_q_time, b_q_time)
  znode = _linear(nodeData, W_kv_node[:, _PERM256], b_kv_node[_PERM256],
                  jnp.bfloat16)
  zedge = _linear(efeat_unique, W_kv_edge[:, _PERM256], b_kv_edge[_PERM256],
                  jnp.bfloat16)
  ztime = _linear(time_unique, W_kv_time[:, _PERM256], b_kv_time[_PERM256],
                  jnp.bfloat16)

  znode, zedge, ztime = _toi32(znode), _toi32(zedge), _toi32(ztime)

  pad = _NP - N
  ndi = jnp.pad(node_dst_inverse, (0, pad)).reshape(1, _NP)
  tdi = jnp.pad(time_dst_inverse, (0, pad)).reshape(1, _NP)
  qour = _qour_sc(ndi, tdi, qn, qt)

  attn, v = _edge_sc(
      idx, node_inverse, efeat_inverse, time_inverse,
      qour, znode, zedge, ztime)
  return (attn, v.reshape(E, H, D_OUT // H))


# speedup vs baseline: 5.7755x; 1.1808x over previous
import dataclasses
import functools

import jax
import jax.numpy as jnp
import numpy as np
from jax.experimental import pallas as pl
from jax.experimental.pallas import tpu as pltpu
from jax.experimental.pallas import tpu_sc as plsc

N = 10000
E = 320000
D_OUT = 128
H = 8

_NP = 10240
_CQ = 128
_CE = 128
_CH = 64

_vector_mesh = plsc.VectorSubcoreMesh(
    core_axis_name="core", subcore_axis_name="subcore")

_sc_params = pltpu.CompilerParams()
if "needs_layout_passes" in pltpu.CompilerParams.__dataclass_fields__:
  _sc_params = dataclasses.replace(_sc_params, needs_layout_passes=False)


def _pack_perms(width):
  q = np.arange(width // 2)
  lo = 32 * (q // 16) + q % 16
  return np.asarray(lo, np.int32), np.asarray(lo + 16, np.int32)


def _linear(x, w, b, out_dtype=jnp.float32, block_rows=512):
  R, K = x.shape
  Dout = w.shape[1]

  def body(x_ref, w_ref, b_ref, o_ref):
    o_ref[...] = (jnp.dot(x_ref[...], w_ref[...],
                          preferred_element_type=jnp.float32)
                  + b_ref[...]).astype(out_dtype)

  br = min(block_rows, R)
  return pl.pallas_call(
      body,
      grid=(pl.cdiv(R, br),),
      in_specs=[
          pl.BlockSpec((br, K), lambda i: (i, 0)),
          pl.BlockSpec((K, Dout), lambda i: (0, 0)),
          pl.BlockSpec((1, Dout), lambda i: (0, 0)),
      ],
      out_specs=pl.BlockSpec((br, Dout), lambda i: (i, 0)),
      out_shape=jax.ShapeDtypeStruct((R, Dout), out_dtype),
  )(x, w, b.reshape(1, Dout))


def _linear_pack(x, w, b, block_rows=512):
  R, K = x.shape
  Dout = w.shape[1]
  lo, hi = _pack_perms(Dout)
  wcat = jnp.concatenate([w[:, lo], w[:, hi]], axis=1)
  bcat = jnp.concatenate([b[lo], b[hi]]).reshape(1, Dout)

  def body(x_ref, w_ref, b_ref, o_ref):
    z = jnp.dot(x_ref[...], w_ref[...],
                preferred_element_type=jnp.float32) + b_ref[...]
    zl = z[:, :Dout // 2].astype(jnp.bfloat16).astype(jnp.float32)
    zh = z[:, Dout // 2:].astype(jnp.bfloat16).astype(jnp.float32)
    ul = jax.lax.bitcast_convert_type(zl, jnp.uint32)
    uh = jax.lax.bitcast_convert_type(zh, jnp.uint32)
    packed = (uh & jnp.uint32(0xFFFF0000)) | (ul >> 16)
    o_ref[...] = jax.lax.bitcast_convert_type(packed, jnp.int32)

  br = min(block_rows, R)
  return pl.pallas_call(
      body,
      grid=(pl.cdiv(R, br),),
      in_specs=[
          pl.BlockSpec((br, K), lambda i: (i, 0)),
          pl.BlockSpec((K, Dout), lambda i: (0, 0)),
          pl.BlockSpec((1, Dout), lambda i: (0, 0)),
      ],
      out_specs=pl.BlockSpec((br, Dout // 2), lambda i: (i, 0)),
      out_shape=jax.ShapeDtypeStruct((R, Dout // 2), jnp.int32),
  )(x, wcat, bcat)


def _qour_sc(ndi, tdi, qn, qt):
  @functools.partial(
      pl.kernel,
      out_type=jax.ShapeDtypeStruct((_NP, D_OUT), jnp.float32),
      mesh=_vector_mesh,
      scratch_types=[pltpu.VMEM((_CQ, D_OUT), jnp.float32)],
      compiler_params=_sc_params,
  )
  def k(ndi_hbm, tdi_hbm, qn_hbm, qt_hbm, out_hbm, qt_s):
    def body(ndi_v, tdi_v, out_v):
      pltpu.sync_copy(qn_hbm.at[ndi_v.at[0]], out_v)
      pltpu.sync_copy(qt_hbm.at[tdi_v.at[0]], qt_s)

      @pl.loop(0, _CQ)
      def _(r):
        for j in range(D_OUT // 16):
          sl = pl.ds(j * 16, 16)
          out_v[r, sl] += qt_s[r, sl]

    pltpu.emit_pipeline(
        body,
        grid=(_NP // _CQ,),
        in_specs=[
            pl.BlockSpec((1, _CQ), lambda i: (0, i)),
            pl.BlockSpec((1, _CQ), lambda i: (0, i)),
        ],
        out_specs=[pl.BlockSpec((_CQ, D_OUT), lambda i: (i, 0))],
        core_axis_name=("core", "subcore"),
        dimension_semantics=(pltpu.PARALLEL,),
    )(ndi_hbm, tdi_hbm, out_hbm)

  return k(ndi, tdi, qn, qt)


_NW = 32
_PW = E // _NW
_C = 40
_SC = 2000
_NS = _PW // _SC
_JC = _SC // _C


def _edge_sc(idx1, ni1, ei1, ti1, qour, zn, ze, zt):
  @functools.partial(
      pl.kernel,
      out_type=(jax.ShapeDtypeStruct((E, H), jnp.float32),
                jax.ShapeDtypeStruct((E, D_OUT), jnp.float32)),
      mesh=_vector_mesh,
      scratch_types=[
          [pltpu.VMEM((_SC,), jnp.int32) for _ in range(4)],
          [pltpu.VMEM((_C, D_OUT), jnp.float32) for _ in range(2)],
          [pltpu.VMEM((_C, D_OUT), jnp.int32) for _ in range(2)],
          [pltpu.VMEM((_C, D_OUT), jnp.int32) for _ in range(2)],
          [pltpu.VMEM((_C, D_OUT), jnp.int32) for _ in range(2)],
          [pltpu.VMEM((_C, D_OUT), jnp.float32) for _ in range(2)],
          [pltpu.VMEM((_C, H), jnp.float32) for _ in range(2)],
          [pltpu.SemaphoreType.DMA for _ in range(2)],
          [pltpu.SemaphoreType.DMA for _ in range(2)],
          pltpu.SemaphoreType.DMA,
      ],
      compiler_params=_sc_params,
  )
  def k(idx_hbm, ni_hbm, ei_hbm, ti_hbm, qour_hbm, zn_hbm, ze_hbm, zt_hbm,
        attn_hbm, v_hbm, ib, qb, znb, zeb, ztb, vb, ab, gsem, osem, isem):
    w = jax.lax.axis_index("subcore") * 2 + jax.lax.axis_index("core")
    base = w * _PW

    def gathers(j, b):
      sl = pl.ds(j * _C, _C)
      return [
          pltpu.make_async_copy(qour_hbm.at[ib[0].at[sl]], qb[b], gsem[b]),
          pltpu.make_async_copy(zn_hbm.at[ib[1].at[sl]], znb[b], gsem[b]),
          pltpu.make_async_copy(ze_hbm.at[ib[2].at[sl]], zeb[b], gsem[b]),
          pltpu.make_async_copy(zt_hbm.at[ib[3].at[sl]], ztb[b], gsem[b]),
      ]

    def outs(c, b):
      sl = pl.ds(base + c * _C, _C)
      return [
          pltpu.make_async_copy(vb[b], v_hbm.at[sl], osem[b]),
          pltpu.make_async_copy(ab[b], attn_hbm.at[sl], osem[b]),
      ]

    lane = jax.lax.iota(jnp.int32, 16)
    prow = lane >> 3
    pcol = lane & 7

    @pl.loop(0, _NS)
    def _(s):
      for src, dst in zip((idx_hbm, ni_hbm, ei_hbm, ti_hbm), ib):
        pltpu.async_copy(src.at[pl.ds(base + s * _SC, _SC)], dst, isem)
      for src, dst in zip((idx_hbm, ni_hbm, ei_hbm, ti_hbm), ib):
        pltpu.make_async_copy(src.at[pl.ds(base + s * _SC, _SC)], dst, isem).wait()

      for d in gathers(0, 0):
        d.start()

      @pl.loop(0, _JC, step=2)
      def _(j0, s=s):
        for b in range(2):
          j = j0 + b
          c = s * _JC + j

          @pl.when(j + 1 < _JC)
          def _(j=j, b=b):
            for d in gathers(j + 1, 1 - b):
              d.start()

          for d in gathers(j, b):
            d.wait()

          @pl.when(c >= 2)
          def _(c=c, b=b):
            for d in outs(c - 2, b):
              d.wait()

          q_s, zn_s, ze_s, zt_s, v_s, a_s = qb[b], znb[b], zeb[b], ztb[b], vb[b], ab[b]

          def unp(x):
            return plsc.unpack(plsc.bitcast(x, jnp.bfloat16),
                               format=plsc.PackFormat.INTERLEAVED)

          @pl.loop(0, _C // 2)
          def _(i, q_s=q_s, zn_s=zn_s, ze_s=ze_s, zt_s=zt_s, v_s=v_s, a_s=a_s):
            acc = jnp.zeros((16,), jnp.float32)
            for e, hbase in ((2 * i, 0), (2 * i + 1, H)):
              for j in range(D_OUT // 32):
                sk = pl.ds(16 * j, 16)
                qa = q_s[e, pl.ds(32 * j, 16)]
                qc = q_s[e, pl.ds(32 * j + 16, 16)]
                na, nc = unp(zn_s[e, sk])
                ea, ec = unp(ze_s[e, sk])
                ta, tc = unp(zt_s[e, sk])
                s0 = jnp.sum(qa * (na + ea + ta))
                s1 = jnp.sum(qc * (nc + ec + tc))
                acc = jnp.where(lane == hbase + 2 * j, s0, acc)
                acc = jnp.where(lane == hbase + 2 * j + 1, s1, acc)
              for j in range(D_OUT // 32):
                sv = pl.ds(D_OUT // 2 + 16 * j, 16)
                na, nc = unp(zn_s[e, sv])
                ea, ec = unp(ze_s[e, sv])
                ta, tc = unp(zt_s[e, sv])
                v_s[e, pl.ds(32 * j, 16)] = na + ea + ta
                v_s[e, pl.ds(32 * j + 16, 16)] = nc + ec + tc
            acc = jnp.where(acc >= 0., acc, 0.2 * acc)
            plsc.store_scatter(a_s, [2 * i + prow, pcol], acc)

          for d in outs(c, b):
            d.start()

    for b in range(2):
      for d in outs(_NS * _JC - 2 + b, b):
        d.wait()

  return k(idx1, ni1, ei1, ti1, qour, zn, ze, zt)


def kernel(idx, nodeData, node_inverse, node_dst_inverse, efeat_unique,
           efeat_inverse, time_unique, time_inverse, time_dst_unique,
           time_dst_inverse, W_q_node, b_q_node, W_q_time, b_q_time,
           W_kv_node, b_kv_node, W_kv_edge, b_kv_edge, W_kv_time, b_kv_time):
  qn = _linear(nodeData, W_q_node, b_q_node)
  qt = _linear(time_dst_unique, W_q_time, b_q_time)
  znode = _linear_pack(nodeData, W_kv_node, b_kv_node)
  zedge = _linear_pack(efeat_unique, W_kv_edge, b_kv_edge)
  ztime = _linear_pack(time_unique, W_kv_time, b_kv_time)

  pad = _NP - N
  ndi = jnp.pad(node_dst_inverse, (0, pad)).reshape(1, _NP)
  tdi = jnp.pad(time_dst_inverse, (0, pad)).reshape(1, _NP)
  qour = _qour_sc(ndi, tdi, qn, qt)

  attn, v = _edge_sc(
      idx, node_inverse, efeat_inverse, time_inverse,
      qour, znode, zedge, ztime)
  return (attn, v.reshape(E, H, D_OUT // H))
